# baseline (speedup 1.0000x reference)
import functools

import jax
import jax.numpy as jnp
import numpy as np
from jax import lax
from jax.experimental import pallas as pl
from jax.experimental.pallas import tpu as pltpu
from jax.experimental.pallas import tpu_sc as plsc

_PERM = (22, 18, 6, 26, 21, 27, 10, 20, 24, 4, 31, 14, 0, 3, 5, 17,
         28, 2, 23, 1, 8, 16, 30, 7, 19, 15, 9, 13, 11, 25, 12, 29)

_BS = 32
_ROW = 160000


def _perm_lookup(w):
    src = jnp.int32(_PERM[0])
    for j in range(1, _BS):
        src = jnp.where(w == j, jnp.int32(_PERM[j]), src)
    return src


_MESH = plsc.VectorSubcoreMesh(core_axis_name="c", subcore_axis_name="s")

_CH = 40000
_CPR = _ROW // _CH
_NBUF = 3


@functools.partial(
    pl.kernel,
    out_type=jax.ShapeDtypeStruct((2 * _BS * _ROW,), jnp.float32),
    mesh=_MESH,
    scratch_types=[
        pltpu.VMEM((_CH,), jnp.float32),
        pltpu.VMEM((_CH,), jnp.float32),
        pltpu.VMEM((_CH,), jnp.float32),
        pltpu.SemaphoreType.DMA,
        pltpu.SemaphoreType.DMA,
        pltpu.SemaphoreType.DMA,
        pltpu.SemaphoreType.DMA,
        pltpu.SemaphoreType.DMA,
        pltpu.SemaphoreType.DMA,
    ],
)
def _remix_sc(src_hbm, out_hbm, buf0, buf1, buf2, ls0, ls1, ls2, ss0, ss1, ss2):
    w = lax.axis_index("s") * 2 + lax.axis_index("c")
    src_row = _perm_lookup(w)

    bufs = (buf0, buf1, buf2)
    lsems = (ls0, ls1, ls2)
    ssems = (ss0, ss1, ss2)

    tasks = []
    for c in range(_CPR):
        tasks.append((src_row * _ROW + c * _CH, w * _ROW + c * _CH))
    for c in range(_CPR):
        off = (_BS + w) * _ROW + c * _CH
        tasks.append((off, off))

    def start_load(i):
        b = i % _NBUF
        return pltpu.async_copy(
            src_hbm.at[pl.ds(tasks[i][0], _CH)], bufs[b], lsems[b])

    def start_store(i):
        b = i % _NBUF
        return pltpu.async_copy(
            bufs[b], out_hbm.at[pl.ds(tasks[i][1], _CH)], ssems[b])

    n = len(tasks)
    loads = [None] * n
    stores = [None] * n
    loads[0] = start_load(0)
    for i in range(n):
        if i + 1 < n:
            if i + 1 >= _NBUF:
                stores[i + 1 - _NBUF].wait()
            loads[i + 1] = start_load(i + 1)
        loads[i].wait()
        stores[i] = start_store(i)
    for i in range(n - _NBUF, n):
        stores[i].wait()


def kernel(sources):
    flat = sources.reshape(2 * _BS * _ROW)
    out = _remix_sc(flat)
    return out.reshape(2, _BS, 1, _ROW)


# speedup vs baseline: 1.0234x; 1.0234x over previous
import functools

import jax
import jax.numpy as jnp
import numpy as np
from jax import lax
from jax.experimental import pallas as pl
from jax.experimental.pallas import tpu as pltpu
from jax.experimental.pallas import tpu_sc as plsc

_PERM = (22, 18, 6, 26, 21, 27, 10, 20, 24, 4, 31, 14, 0, 3, 5, 17,
         28, 2, 23, 1, 8, 16, 30, 7, 19, 15, 9, 13, 11, 25, 12, 29)

_BS = 32
_ROW = 160000


def _perm_lookup(w):
    src = jnp.int32(_PERM[0])
    for j in range(1, _BS):
        src = jnp.where(w == j, jnp.int32(_PERM[j]), src)
    return src


_MESH = plsc.VectorSubcoreMesh(core_axis_name="c", subcore_axis_name="s")

_CH = 40000
_CPR = _ROW // _CH
_NBUF = 2


@functools.partial(
    pl.kernel,
    out_type=jax.ShapeDtypeStruct((2 * _BS * _ROW,), jnp.float32),
    mesh=_MESH,
    scratch_types=[
        pltpu.VMEM((_CH,), jnp.float32),
        pltpu.VMEM((_CH,), jnp.float32),
        pltpu.SemaphoreType.DMA,
        pltpu.SemaphoreType.DMA,
        pltpu.SemaphoreType.DMA,
        pltpu.SemaphoreType.DMA,
    ],
)
def _remix_sc(src_hbm, out_hbm, buf0, buf1, ls0, ls1, ss0, ss1):
    w = lax.axis_index("s") * 2 + lax.axis_index("c")
    src_row = _perm_lookup(w)

    bufs = (buf0, buf1)
    lsems = (ls0, ls1)
    ssems = (ss0, ss1)

    tasks = []
    for c in range(_CPR):
        tasks.append((src_row * _ROW + c * _CH, w * _ROW + c * _CH))
    for c in range(_CPR):
        off = (_BS + w) * _ROW + c * _CH
        tasks.append((off, off))

    def start_load(i):
        b = i % _NBUF
        return pltpu.async_copy(
            src_hbm.at[pl.ds(tasks[i][0], _CH)], bufs[b], lsems[b])

    def start_store(i):
        b = i % _NBUF
        return pltpu.async_copy(
            bufs[b], out_hbm.at[pl.ds(tasks[i][1], _CH)], ssems[b])

    n = len(tasks)
    loads = [None] * n
    stores = [None] * n
    loads[0] = start_load(0)
    for i in range(n):
        if i + 1 < n:
            if i + 1 >= _NBUF:
                stores[i + 1 - _NBUF].wait()
            loads[i + 1] = start_load(i + 1)
        loads[i].wait()
        stores[i] = start_store(i)
    for i in range(n - _NBUF, n):
        stores[i].wait()


def kernel(sources):
    flat = sources.reshape(2 * _BS * _ROW)
    out = _remix_sc(flat)
    return out.reshape(2, _BS, 1, _ROW)


# speedup vs baseline: 1.0278x; 1.0043x over previous
import functools

import jax
import jax.numpy as jnp
import numpy as np
from jax import lax
from jax.experimental import pallas as pl
from jax.experimental.pallas import tpu as pltpu
from jax.experimental.pallas import tpu_sc as plsc

_PERM = (22, 18, 6, 26, 21, 27, 10, 20, 24, 4, 31, 14, 0, 3, 5, 17,
         28, 2, 23, 1, 8, 16, 30, 7, 19, 15, 9, 13, 11, 25, 12, 29)

_BS = 32
_ROW = 160000


def _perm_lookup(w):
    src = jnp.int32(_PERM[0])
    for j in range(1, _BS):
        src = jnp.where(w == j, jnp.int32(_PERM[j]), src)
    return src


_MESH = plsc.VectorSubcoreMesh(core_axis_name="c", subcore_axis_name="s")

_CH = 20000
_CPR = _ROW // _CH
_NBUF = 4


@functools.partial(
    pl.kernel,
    out_type=jax.ShapeDtypeStruct((2 * _BS * _ROW,), jnp.float32),
    mesh=_MESH,
    scratch_types=[
        pltpu.VMEM((_CH,), jnp.float32),
        pltpu.VMEM((_CH,), jnp.float32),
        pltpu.VMEM((_CH,), jnp.float32),
        pltpu.VMEM((_CH,), jnp.float32),
        pltpu.SemaphoreType.DMA,
        pltpu.SemaphoreType.DMA,
        pltpu.SemaphoreType.DMA,
        pltpu.SemaphoreType.DMA,
        pltpu.SemaphoreType.DMA,
        pltpu.SemaphoreType.DMA,
        pltpu.SemaphoreType.DMA,
        pltpu.SemaphoreType.DMA,
    ],
)
def _remix_sc(src_hbm, out_hbm, buf0, buf1, buf2, buf3,
              ls0, ls1, ls2, ls3, ss0, ss1, ss2, ss3):
    w = lax.axis_index("s") * 2 + lax.axis_index("c")
    src_row = _perm_lookup(w)

    bufs = (buf0, buf1, buf2, buf3)
    lsems = (ls0, ls1, ls2, ls3)
    ssems = (ss0, ss1, ss2, ss3)

    tasks = []
    for c in range(_CPR):
        tasks.append((src_row * _ROW + c * _CH, w * _ROW + c * _CH))
    for c in range(_CPR):
        off = (_BS + w) * _ROW + c * _CH
        tasks.append((off, off))

    def start_load(i):
        b = i % _NBUF
        return pltpu.async_copy(
            src_hbm.at[pl.ds(tasks[i][0], _CH)], bufs[b], lsems[b])

    def start_store(i):
        b = i % _NBUF
        return pltpu.async_copy(
            bufs[b], out_hbm.at[pl.ds(tasks[i][1], _CH)], ssems[b])

    n = len(tasks)
    loads = [None] * n
    stores = [None] * n
    for j in range(_NBUF - 1):
        loads[j] = start_load(j)
    for i in range(n):
        j = i + _NBUF - 1
        if j < n:
            if j >= _NBUF:
                stores[j - _NBUF].wait()
            loads[j] = start_load(j)
        loads[i].wait()
        stores[i] = start_store(i)
    for i in range(n - _NBUF, n):
        stores[i].wait()


def kernel(sources):
    flat = sources.reshape(2 * _BS * _ROW)
    out = _remix_sc(flat)
    return out.reshape(2, _BS, 1, _ROW)
